# baseline (speedup 1.0000x reference)
import functools

import jax
import jax.numpy as jnp
from jax import lax
from jax.experimental import pallas as pl
from jax.experimental.pallas import tpu as pltpu
from jax.experimental.pallas import tpu_sc as plsc

_NC = 2
_NS = 16
_LANES = 128
_BATCH = 128


@functools.partial(jax.jit, static_argnames=("n", "nchunks", "nb"))
def _segsum(h2, src_all, dstr, zeros, *, n, nchunks, nb):
  cpc = nchunks // _NC
  npad = -(-n // (_NS * 8)) * (_NS * 8)
  rpt = npad // _NS

  mesh = plsc.VectorSubcoreMesh(core_axis_name="c", subcore_axis_name="s")

  @functools.partial(
      pl.kernel,
      out_type=jax.ShapeDtypeStruct((nchunks * npad, _LANES), jnp.float32),
      mesh=mesh,
      scratch_types=[
          pltpu.VMEM((nb, _BATCH), jnp.int32),
          pltpu.VMEM((nb, _BATCH), jnp.int32),
          pltpu.VMEM((_BATCH, _LANES), jnp.float32),
          pltpu.VMEM_SHARED((npad, _LANES), jnp.float32),
          pltpu.SemaphoreType.DMA,
      ],
  )
  def seg_kernel(h_hbm, src_hbm, dst_hbm, zero_hbm, out_hbm,
                 src_v, dst_v, rows_v, acc, sem):
    c = lax.axis_index("c")
    s = lax.axis_index("s")
    pltpu.sync_copy(dst_hbm.at[s], dst_v)
    for ci in range(cpc):
      chunk = c + _NC * ci
      pltpu.sync_copy(src_hbm.at[chunk * _NS + s], src_v)
      acc_off = pl.multiple_of(s * rpt, 8)
      pltpu.sync_copy(zero_hbm, acc.at[pl.ds(acc_off, rpt)])
      plsc.subcore_barrier()

      def batch_body(b, carry):
        pltpu.async_copy(h_hbm.at[src_v.at[b]], rows_v, sem).wait()
        pltpu.sync_copy(rows_v, acc.at[dst_v.at[b]], add=True)
        return carry

      lax.fori_loop(0, nb, batch_body, 0)
      plsc.subcore_barrier()
      out_off = pl.multiple_of(chunk * npad + s * rpt, 8)
      pltpu.sync_copy(acc.at[pl.ds(acc_off, rpt)],
                      out_hbm.at[pl.ds(out_off, rpt)])

  return seg_kernel(h2, src_all, dstr, zeros)


_BN = 2000


def _mm_bias(x, w, b):
  n, k = x.shape
  _, m = w.shape
  r = n // _BN

  def kern(x_ref, w_ref, b_ref, o_ref):
    o_ref[...] = jnp.dot(x_ref[...], w_ref[...],
                         preferred_element_type=jnp.float32) + b_ref[...]

  return pl.pallas_call(
      kern,
      grid=(r,),
      in_specs=[
          pl.BlockSpec((_BN, k), lambda i: (i, 0)),
          pl.BlockSpec((k, m), lambda i: (0, 0)),
          pl.BlockSpec((1, m), lambda i: (0, 0)),
      ],
      out_specs=pl.BlockSpec((_BN, m), lambda i: (i, 0)),
      out_shape=jax.ShapeDtypeStruct((n, m), jnp.float32),
  )(x, w, b)


def _mm1_stats(h, aggc, eps, w, b):
  n, din = h.shape
  _, m = w.shape
  kc = din // _LANES
  r = n // _BN

  def kern(h_ref, a_ref, e_ref, w_ref, b_ref, y_ref, s1_ref, s2_ref):
    i = pl.program_id(0)
    k = pl.program_id(1)
    z = (1.0 + e_ref[0, 0]) * h_ref[...] + a_ref[0]
    part = jnp.dot(z, w_ref[...], preferred_element_type=jnp.float32)

    @pl.when(k == 0)
    def _():
      y_ref[...] = part + b_ref[...]

    @pl.when(k > 0)
    def _():
      y_ref[...] += part

    @pl.when(k == kc - 1)
    def _():
      y = y_ref[...]
      cs = jnp.sum(y, axis=0, keepdims=True)
      cq = jnp.sum(y * y, axis=0, keepdims=True)

      @pl.when(i == 0)
      def _():
        s1_ref[...] = cs
        s2_ref[...] = cq

      @pl.when(i > 0)
      def _():
        s1_ref[...] += cs
        s2_ref[...] += cq

  return pl.pallas_call(
      kern,
      grid=(r, kc),
      in_specs=[
          pl.BlockSpec((_BN, _LANES), lambda i, k: (i, k)),
          pl.BlockSpec((1, _BN, _LANES), lambda i, k: (k, i, 0)),
          pl.BlockSpec((1, 1), lambda i, k: (0, 0)),
          pl.BlockSpec((_LANES, m), lambda i, k: (k, 0)),
          pl.BlockSpec((1, m), lambda i, k: (0, 0)),
      ],
      out_specs=[
          pl.BlockSpec((_BN, m), lambda i, k: (i, 0)),
          pl.BlockSpec((1, m), lambda i, k: (0, 0)),
          pl.BlockSpec((1, m), lambda i, k: (0, 0)),
      ],
      out_shape=[
          jax.ShapeDtypeStruct((n, m), jnp.float32),
          jax.ShapeDtypeStruct((1, m), jnp.float32),
          jax.ShapeDtypeStruct((1, m), jnp.float32),
      ],
  )(h, aggc, eps, w, b)


def _bn_mm_stats(y, s1, s2, g, bb, w, b):
  n, _ = y.shape
  k, m = w.shape
  r = n // _BN

  def kern(y_ref, s1_ref, s2_ref, g_ref, bb_ref, w_ref, b_ref,
           q_ref, q1_ref, q2_ref):
    i = pl.program_id(0)
    mean = s1_ref[...] / n
    var = s2_ref[...] / n - mean * mean
    inv = lax.rsqrt(var + 1e-5)
    t = jnp.maximum((y_ref[...] - mean) * (inv * g_ref[...]) + bb_ref[...],
                    0.0)
    q = jnp.dot(t, w_ref[...], preferred_element_type=jnp.float32) + b_ref[...]
    q_ref[...] = q
    cs = jnp.sum(q, axis=0, keepdims=True)
    cq = jnp.sum(q * q, axis=0, keepdims=True)

    @pl.when(i == 0)
    def _():
      q1_ref[...] = cs
      q2_ref[...] = cq

    @pl.when(i > 0)
    def _():
      q1_ref[...] += cs
      q2_ref[...] += cq

  return pl.pallas_call(
      kern,
      grid=(r,),
      in_specs=[
          pl.BlockSpec((_BN, k), lambda i: (i, 0)),
          pl.BlockSpec((1, k), lambda i: (0, 0)),
          pl.BlockSpec((1, k), lambda i: (0, 0)),
          pl.BlockSpec((1, k), lambda i: (0, 0)),
          pl.BlockSpec((1, k), lambda i: (0, 0)),
          pl.BlockSpec((k, m), lambda i: (0, 0)),
          pl.BlockSpec((1, m), lambda i: (0, 0)),
      ],
      out_specs=[
          pl.BlockSpec((_BN, m), lambda i: (i, 0)),
          pl.BlockSpec((1, m), lambda i: (0, 0)),
          pl.BlockSpec((1, m), lambda i: (0, 0)),
      ],
      out_shape=[
          jax.ShapeDtypeStruct((n, m), jnp.float32),
          jax.ShapeDtypeStruct((1, m), jnp.float32),
          jax.ShapeDtypeStruct((1, m), jnp.float32),
      ],
  )(y, s1, s2, g, bb, w, b)


def _bn_stats(q, s1, s2, g, bb):
  n, m = q.shape
  r = n // _BN

  def kern(q_ref, s1_ref, s2_ref, g_ref, bb_ref, t_ref, t1_ref, t2_ref):
    i = pl.program_id(0)
    mean = s1_ref[...] / n
    var = s2_ref[...] / n - mean * mean
    inv = lax.rsqrt(var + 1e-5)
    t = jnp.maximum((q_ref[...] - mean) * (inv * g_ref[...]) + bb_ref[...],
                    0.0)
    t_ref[...] = t
    cs = jnp.sum(t, axis=0, keepdims=True)
    cq = jnp.sum(t * t, axis=0, keepdims=True)

    @pl.when(i == 0)
    def _():
      t1_ref[...] = cs
      t2_ref[...] = cq

    @pl.when(i > 0)
    def _():
      t1_ref[...] += cs
      t2_ref[...] += cq

  return pl.pallas_call(
      kern,
      grid=(r,),
      in_specs=[
          pl.BlockSpec((_BN, m), lambda i: (i, 0)),
          pl.BlockSpec((1, m), lambda i: (0, 0)),
          pl.BlockSpec((1, m), lambda i: (0, 0)),
          pl.BlockSpec((1, m), lambda i: (0, 0)),
          pl.BlockSpec((1, m), lambda i: (0, 0)),
      ],
      out_specs=[
          pl.BlockSpec((_BN, m), lambda i: (i, 0)),
          pl.BlockSpec((1, m), lambda i: (0, 0)),
          pl.BlockSpec((1, m), lambda i: (0, 0)),
      ],
      out_shape=[
          jax.ShapeDtypeStruct((n, m), jnp.float32),
          jax.ShapeDtypeStruct((1, m), jnp.float32),
          jax.ShapeDtypeStruct((1, m), jnp.float32),
      ],
  )(q, s1, s2, g, bb)


def _bn_pred(t, s1, s2, g, bb, pw, score_in):
  n, m = t.shape
  _, mo = pw.shape
  r = n // _BN

  def kern(t_ref, s1_ref, s2_ref, g_ref, bb_ref, pw_ref, sc_ref,
           h_ref, so_ref):
    mean = s1_ref[...] / n
    var = s2_ref[...] / n - mean * mean
    inv = lax.rsqrt(var + 1e-5)
    h = jnp.maximum((t_ref[...] - mean) * (inv * g_ref[...]) + bb_ref[...],
                    0.0)
    h_ref[...] = h
    so_ref[...] = sc_ref[...] + jnp.dot(h, pw_ref[...],
                                        preferred_element_type=jnp.float32)

  return pl.pallas_call(
      kern,
      grid=(r,),
      in_specs=[
          pl.BlockSpec((_BN, m), lambda i: (i, 0)),
          pl.BlockSpec((1, m), lambda i: (0, 0)),
          pl.BlockSpec((1, m), lambda i: (0, 0)),
          pl.BlockSpec((1, m), lambda i: (0, 0)),
          pl.BlockSpec((1, m), lambda i: (0, 0)),
          pl.BlockSpec((m, mo), lambda i: (0, 0)),
          pl.BlockSpec((_BN, mo), lambda i: (i, 0)),
      ],
      out_specs=[
          pl.BlockSpec((_BN, m), lambda i: (i, 0)),
          pl.BlockSpec((_BN, mo), lambda i: (i, 0)),
      ],
      out_shape=[
          jax.ShapeDtypeStruct((n, m), jnp.float32),
          jax.ShapeDtypeStruct((n, mo), jnp.float32),
      ],
      input_output_aliases={6: 1},
  )(t, s1, s2, g, bb, pw, score_in)


def kernel(h, edge_index, params):
  n, din0 = h.shape
  e = edge_index.shape[1]

  nb = -(-e // (_NS * _BATCH * 16)) * 16
  ep = _NS * nb * _BATCH
  src = jnp.concatenate(
      [edge_index[0], jnp.zeros((ep - e,), jnp.int32)]).reshape(_NS, nb,
                                                                _BATCH)
  dst = jnp.concatenate(
      [edge_index[1], jnp.full((ep - e,), n, jnp.int32)]).reshape(_NS, nb,
                                                                  _BATCH)
  npad = -(-n // (_NS * 8)) * (_NS * 8)
  zeros = jnp.zeros((npad // _NS, _LANES), jnp.float32)

  def offset_src(nchunks):
    off = (jnp.arange(nchunks, dtype=jnp.int32) * n)[:, None, None, None]
    return (src[None] + off).reshape(nchunks * _NS, nb, _BATCH)

  src_by_nc = {}
  for i in range(3):
    nch = (din0 if i == 0 else 512) // _LANES
    if nch not in src_by_nc:
      src_by_nc[nch] = offset_src(nch)


  b_total = (params["pred0_b"] + params["pred1_b"] + params["pred2_b"]
             + params["pred3_b"]).reshape(1, -1)
  score = _mm_bias(h, params["pred0_W"], b_total)

  hcur = h
  for i in range(3):
    d = hcur.shape[1]
    nch = d // _LANES
    h2 = jnp.moveaxis(hcur.reshape(n, nch, _LANES), 1, 0).reshape(
        nch * n, _LANES)
    aggf = _segsum(h2, src_by_nc[nch], dst, zeros,
                   n=n, nchunks=nch, nb=nb)
    aggc = aggf.reshape(nch, npad, _LANES)

    eps = params[f"eps{i}"].reshape(1, 1)
    y, s1, s2 = _mm1_stats(hcur, aggc, eps,
                           params[f"mlp{i}_W1"],
                           params[f"mlp{i}_b1"].reshape(1, -1))
    q, q1, q2 = _bn_mm_stats(y, s1, s2,
                             params[f"mlp{i}_bng"].reshape(1, -1),
                             params[f"mlp{i}_bnb"].reshape(1, -1),
                             params[f"mlp{i}_W2"],
                             params[f"mlp{i}_b2"].reshape(1, -1))
    t, t1, t2 = _bn_stats(q, q1, q2,
                          params[f"apply{i}_bng"].reshape(1, -1),
                          params[f"apply{i}_bnb"].reshape(1, -1))
    hcur, score = _bn_pred(t, t1, t2,
                           params[f"out{i}_bng"].reshape(1, -1),
                           params[f"out{i}_bnb"].reshape(1, -1),
                           params[f"pred{i + 1}_W"], score)

  return score


# speedup vs baseline: 1.3389x; 1.3389x over previous
import functools

import jax
import jax.numpy as jnp
from jax import lax
from jax.experimental import pallas as pl
from jax.experimental.pallas import tpu as pltpu
from jax.experimental.pallas import tpu_sc as plsc

_NC = 2
_NS = 16
_LANES = 128
_BATCH = 128


@functools.partial(jax.jit, static_argnames=("n", "nchunks", "nb"))
def _segsum(h2, src_all, dstr, zeros, *, n, nchunks, nb):
  cpc = nchunks // _NC
  npad = -(-n // (_NS * 8)) * (_NS * 8)
  rpt = npad // _NS

  mesh = plsc.VectorSubcoreMesh(core_axis_name="c", subcore_axis_name="s")

  @functools.partial(
      pl.kernel,
      out_type=jax.ShapeDtypeStruct((nchunks * npad, _LANES), jnp.float32),
      mesh=mesh,
      scratch_types=[
          pltpu.VMEM((nb, _BATCH), jnp.int32),
          pltpu.VMEM((nb, _BATCH), jnp.int32),
          pltpu.VMEM((_BATCH, _LANES), jnp.float32),
          pltpu.VMEM_SHARED((npad, _LANES), jnp.float32),
          pltpu.SemaphoreType.DMA,
      ],
  )
  def seg_kernel(h_hbm, src_hbm, dst_hbm, zero_hbm, out_hbm,
                 src_v, dst_v, rows_v, acc, sem):
    c = lax.axis_index("c")
    s = lax.axis_index("s")
    pltpu.sync_copy(dst_hbm.at[s], dst_v)
    for ci in range(cpc):
      chunk = c + _NC * ci
      pltpu.sync_copy(src_hbm.at[chunk * _NS + s], src_v)
      acc_off = pl.multiple_of(s * rpt, 8)
      pltpu.sync_copy(zero_hbm, acc.at[pl.ds(acc_off, rpt)])
      plsc.subcore_barrier()

      def batch_body(b, carry):
        pltpu.async_copy(h_hbm.at[src_v.at[b]], rows_v, sem).wait()
        pltpu.sync_copy(rows_v, acc.at[dst_v.at[b]], add=True)
        return carry

      lax.fori_loop(0, nb, batch_body, 0)
      plsc.subcore_barrier()
      out_off = pl.multiple_of(chunk * npad + s * rpt, 8)
      pltpu.sync_copy(acc.at[pl.ds(acc_off, rpt)],
                      out_hbm.at[pl.ds(out_off, rpt)])

  return seg_kernel(h2, src_all, dstr, zeros)


_BN = 2000


def _mm_bias(x, w, b):
  n, k = x.shape
  _, m = w.shape
  r = n // _BN

  def kern(x_ref, w_ref, b_ref, o_ref):
    o_ref[...] = jnp.dot(x_ref[...], w_ref[...],
                         preferred_element_type=jnp.float32) + b_ref[...]

  return pl.pallas_call(
      kern,
      grid=(r,),
      in_specs=[
          pl.BlockSpec((_BN, k), lambda i: (i, 0)),
          pl.BlockSpec((k, m), lambda i: (0, 0)),
          pl.BlockSpec((1, m), lambda i: (0, 0)),
      ],
      out_specs=pl.BlockSpec((_BN, m), lambda i: (i, 0)),
      out_shape=jax.ShapeDtypeStruct((n, m), jnp.float32),
  )(x, w, b)


def _mm1_stats(h, aggc, eps, w, b):
  n, din = h.shape
  _, m = w.shape
  kc = din // _LANES
  r = n // _BN

  def kern(h_ref, a_ref, e_ref, w_ref, b_ref, y_ref, s1_ref, s2_ref):
    i = pl.program_id(0)
    k = pl.program_id(1)
    z = (1.0 + e_ref[0, 0]) * h_ref[...] + a_ref[0]
    part = jnp.dot(z, w_ref[...], preferred_element_type=jnp.float32)

    @pl.when(k == 0)
    def _():
      y_ref[...] = part + b_ref[...]

    @pl.when(k > 0)
    def _():
      y_ref[...] += part

    @pl.when(k == kc - 1)
    def _():
      y = y_ref[...]
      cs = jnp.sum(y, axis=0, keepdims=True)
      cq = jnp.sum(y * y, axis=0, keepdims=True)

      @pl.when(i == 0)
      def _():
        s1_ref[...] = cs
        s2_ref[...] = cq

      @pl.when(i > 0)
      def _():
        s1_ref[...] += cs
        s2_ref[...] += cq

  return pl.pallas_call(
      kern,
      grid=(r, kc),
      in_specs=[
          pl.BlockSpec((_BN, _LANES), lambda i, k: (i, k)),
          pl.BlockSpec((1, _BN, _LANES), lambda i, k: (k, i, 0)),
          pl.BlockSpec((1, 1), lambda i, k: (0, 0)),
          pl.BlockSpec((_LANES, m), lambda i, k: (k, 0)),
          pl.BlockSpec((1, m), lambda i, k: (0, 0)),
      ],
      out_specs=[
          pl.BlockSpec((_BN, m), lambda i, k: (i, 0)),
          pl.BlockSpec((1, m), lambda i, k: (0, 0)),
          pl.BlockSpec((1, m), lambda i, k: (0, 0)),
      ],
      out_shape=[
          jax.ShapeDtypeStruct((n, m), jnp.float32),
          jax.ShapeDtypeStruct((1, m), jnp.float32),
          jax.ShapeDtypeStruct((1, m), jnp.float32),
      ],
  )(h, aggc, eps, w, b)


def _bn_mm_stats(y, s1, s2, g, bb, w, b):
  n, _ = y.shape
  k, m = w.shape
  r = n // _BN

  def kern(y_ref, s1_ref, s2_ref, g_ref, bb_ref, w_ref, b_ref,
           q_ref, q1_ref, q2_ref):
    i = pl.program_id(0)
    mean = s1_ref[...] / n
    var = s2_ref[...] / n - mean * mean
    inv = lax.rsqrt(var + 1e-5)
    t = jnp.maximum((y_ref[...] - mean) * (inv * g_ref[...]) + bb_ref[...],
                    0.0)
    q = jnp.dot(t, w_ref[...], preferred_element_type=jnp.float32) + b_ref[...]
    q_ref[...] = q
    cs = jnp.sum(q, axis=0, keepdims=True)
    cq = jnp.sum(q * q, axis=0, keepdims=True)

    @pl.when(i == 0)
    def _():
      q1_ref[...] = cs
      q2_ref[...] = cq

    @pl.when(i > 0)
    def _():
      q1_ref[...] += cs
      q2_ref[...] += cq

  return pl.pallas_call(
      kern,
      grid=(r,),
      in_specs=[
          pl.BlockSpec((_BN, k), lambda i: (i, 0)),
          pl.BlockSpec((1, k), lambda i: (0, 0)),
          pl.BlockSpec((1, k), lambda i: (0, 0)),
          pl.BlockSpec((1, k), lambda i: (0, 0)),
          pl.BlockSpec((1, k), lambda i: (0, 0)),
          pl.BlockSpec((k, m), lambda i: (0, 0)),
          pl.BlockSpec((1, m), lambda i: (0, 0)),
      ],
      out_specs=[
          pl.BlockSpec((_BN, m), lambda i: (i, 0)),
          pl.BlockSpec((1, m), lambda i: (0, 0)),
          pl.BlockSpec((1, m), lambda i: (0, 0)),
      ],
      out_shape=[
          jax.ShapeDtypeStruct((n, m), jnp.float32),
          jax.ShapeDtypeStruct((1, m), jnp.float32),
          jax.ShapeDtypeStruct((1, m), jnp.float32),
      ],
  )(y, s1, s2, g, bb, w, b)


def _bn_stats(q, s1, s2, g, bb):
  n, m = q.shape
  r = n // _BN

  def kern(q_ref, s1_ref, s2_ref, g_ref, bb_ref, t_ref, t1_ref, t2_ref):
    i = pl.program_id(0)
    mean = s1_ref[...] / n
    var = s2_ref[...] / n - mean * mean
    inv = lax.rsqrt(var + 1e-5)
    t = jnp.maximum((q_ref[...] - mean) * (inv * g_ref[...]) + bb_ref[...],
                    0.0)
    t_ref[...] = t
    cs = jnp.sum(t, axis=0, keepdims=True)
    cq = jnp.sum(t * t, axis=0, keepdims=True)

    @pl.when(i == 0)
    def _():
      t1_ref[...] = cs
      t2_ref[...] = cq

    @pl.when(i > 0)
    def _():
      t1_ref[...] += cs
      t2_ref[...] += cq

  return pl.pallas_call(
      kern,
      grid=(r,),
      in_specs=[
          pl.BlockSpec((_BN, m), lambda i: (i, 0)),
          pl.BlockSpec((1, m), lambda i: (0, 0)),
          pl.BlockSpec((1, m), lambda i: (0, 0)),
          pl.BlockSpec((1, m), lambda i: (0, 0)),
          pl.BlockSpec((1, m), lambda i: (0, 0)),
      ],
      out_specs=[
          pl.BlockSpec((_BN, m), lambda i: (i, 0)),
          pl.BlockSpec((1, m), lambda i: (0, 0)),
          pl.BlockSpec((1, m), lambda i: (0, 0)),
      ],
      out_shape=[
          jax.ShapeDtypeStruct((n, m), jnp.float32),
          jax.ShapeDtypeStruct((1, m), jnp.float32),
          jax.ShapeDtypeStruct((1, m), jnp.float32),
      ],
  )(q, s1, s2, g, bb)


def _bn_pred(t, s1, s2, g, bb, pw, score_in):
  n, m = t.shape
  _, mo = pw.shape
  r = n // _BN

  def kern(t_ref, s1_ref, s2_ref, g_ref, bb_ref, pw_ref, sc_ref,
           h_ref, so_ref):
    mean = s1_ref[...] / n
    var = s2_ref[...] / n - mean * mean
    inv = lax.rsqrt(var + 1e-5)
    h = jnp.maximum((t_ref[...] - mean) * (inv * g_ref[...]) + bb_ref[...],
                    0.0)
    h_ref[...] = h
    so_ref[...] = sc_ref[...] + jnp.dot(h, pw_ref[...],
                                        preferred_element_type=jnp.float32)

  return pl.pallas_call(
      kern,
      grid=(r,),
      in_specs=[
          pl.BlockSpec((_BN, m), lambda i: (i, 0)),
          pl.BlockSpec((1, m), lambda i: (0, 0)),
          pl.BlockSpec((1, m), lambda i: (0, 0)),
          pl.BlockSpec((1, m), lambda i: (0, 0)),
          pl.BlockSpec((1, m), lambda i: (0, 0)),
          pl.BlockSpec((m, mo), lambda i: (0, 0)),
          pl.BlockSpec((_BN, mo), lambda i: (i, 0)),
      ],
      out_specs=[
          pl.BlockSpec((_BN, m), lambda i: (i, 0)),
          pl.BlockSpec((_BN, mo), lambda i: (i, 0)),
      ],
      out_shape=[
          jax.ShapeDtypeStruct((n, m), jnp.float32),
          jax.ShapeDtypeStruct((n, mo), jnp.float32),
      ],
      input_output_aliases={6: 1},
  )(t, s1, s2, g, bb, pw, score_in)


def kernel(h, edge_index, params):
  n, din0 = h.shape
  e = edge_index.shape[1]

  nb = -(-e // (_NS * _BATCH))
  ep = _NS * nb * _BATCH
  src = jnp.concatenate(
      [edge_index[0], jnp.zeros((ep - e,), jnp.int32)]).reshape(_NS, nb,
                                                                _BATCH)
  dst = jnp.concatenate(
      [edge_index[1], jnp.full((ep - e,), n, jnp.int32)]).reshape(_NS, nb,
                                                                  _BATCH)
  npad = -(-n // (_NS * 8)) * (_NS * 8)
  zeros = jnp.zeros((npad // _NS, _LANES), jnp.float32)

  def offset_src(nchunks):
    off = (jnp.arange(nchunks, dtype=jnp.int32) * n)[:, None, None, None]
    return (src[None] + off).reshape(nchunks * _NS, nb, _BATCH)

  src_by_nc = {}
  for i in range(3):
    nch = (din0 if i == 0 else 512) // _LANES
    if nch not in src_by_nc:
      src_by_nc[nch] = offset_src(nch)


  b_total = (params["pred0_b"] + params["pred1_b"] + params["pred2_b"]
             + params["pred3_b"]).reshape(1, -1)
  score = _mm_bias(h, params["pred0_W"], b_total)

  hcur = h
  for i in range(3):
    d = hcur.shape[1]
    nch = d // _LANES
    h2 = jnp.moveaxis(hcur.reshape(n, nch, _LANES), 1, 0).reshape(
        nch * n, _LANES)
    aggf = _segsum(h2, src_by_nc[nch], dst, zeros,
                   n=n, nchunks=nch, nb=nb)
    aggc = aggf.reshape(nch, npad, _LANES)

    eps = params[f"eps{i}"].reshape(1, 1)
    y, s1, s2 = _mm1_stats(hcur, aggc, eps,
                           params[f"mlp{i}_W1"],
                           params[f"mlp{i}_b1"].reshape(1, -1))
    q, q1, q2 = _bn_mm_stats(y, s1, s2,
                             params[f"mlp{i}_bng"].reshape(1, -1),
                             params[f"mlp{i}_bnb"].reshape(1, -1),
                             params[f"mlp{i}_W2"],
                             params[f"mlp{i}_b2"].reshape(1, -1))
    t, t1, t2 = _bn_stats(q, q1, q2,
                          params[f"apply{i}_bng"].reshape(1, -1),
                          params[f"apply{i}_bnb"].reshape(1, -1))
    hcur, score = _bn_pred(t, t1, t2,
                           params[f"out{i}_bng"].reshape(1, -1),
                           params[f"out{i}_bnb"].reshape(1, -1),
                           params[f"pred{i + 1}_W"], score)

  return score


# speedup vs baseline: 1.7992x; 1.3438x over previous
import functools

import jax
import jax.numpy as jnp
from jax import lax
from jax.experimental import pallas as pl
from jax.experimental.pallas import tpu as pltpu
from jax.experimental.pallas import tpu_sc as plsc

_NC = 2
_NS = 16
_LANES = 128
_BATCH = 128


@functools.partial(jax.jit, static_argnames=("n", "nchunks", "nb"))
def _segsum(h2, src_all, dstr, zeros, *, n, nchunks, nb):
  cpc = nchunks // _NC
  npad = -(-n // (_NS * 8)) * (_NS * 8)
  rpt = npad // _NS

  mesh = plsc.VectorSubcoreMesh(core_axis_name="c", subcore_axis_name="s")

  @functools.partial(
      pl.kernel,
      out_type=jax.ShapeDtypeStruct((nchunks * npad, _LANES), jnp.float32),
      mesh=mesh,
      scratch_types=[
          pltpu.VMEM((nb, _BATCH), jnp.int32),
          pltpu.VMEM((nb, _BATCH), jnp.int32),
          pltpu.VMEM((_BATCH, _LANES), jnp.float32),
          pltpu.VMEM_SHARED((npad, _LANES), jnp.float32),
          pltpu.SemaphoreType.DMA,
      ],
  )
  def seg_kernel(h_hbm, src_hbm, dst_hbm, zero_hbm, out_hbm,
                 src_v, dst_v, rows_v, acc, sem):
    c = lax.axis_index("c")
    s = lax.axis_index("s")
    pltpu.sync_copy(dst_hbm.at[s], dst_v)
    for ci in range(cpc):
      chunk = c + _NC * ci
      pltpu.sync_copy(src_hbm.at[chunk * _NS + s], src_v)
      acc_off = pl.multiple_of(s * rpt, 8)
      pltpu.sync_copy(zero_hbm, acc.at[pl.ds(acc_off, rpt)])
      plsc.subcore_barrier()

      def batch_body(b, carry):
        pltpu.async_copy(h_hbm.at[src_v.at[b]], rows_v, sem).wait()
        pltpu.sync_copy(rows_v, acc.at[dst_v.at[b]], add=True)
        return carry

      lax.fori_loop(0, nb, batch_body, 0)
      plsc.subcore_barrier()
      out_off = pl.multiple_of(chunk * npad + s * rpt, 8)
      pltpu.sync_copy(acc.at[pl.ds(acc_off, rpt)],
                      out_hbm.at[pl.ds(out_off, rpt)])

  return seg_kernel(h2, src_all, dstr, zeros)


_BN = 2000


def _mm_bias(x, w, b):
  n, k = x.shape
  _, m = w.shape
  r = n // _BN

  def kern(x_ref, w_ref, b_ref, o_ref):
    o_ref[...] = jnp.dot(x_ref[...], w_ref[...],
                         preferred_element_type=jnp.float32) + b_ref[...]

  return pl.pallas_call(
      kern,
      grid=(r,),
      in_specs=[
          pl.BlockSpec((_BN, k), lambda i: (i, 0)),
          pl.BlockSpec((k, m), lambda i: (0, 0)),
          pl.BlockSpec((1, m), lambda i: (0, 0)),
      ],
      out_specs=pl.BlockSpec((_BN, m), lambda i: (i, 0)),
      out_shape=jax.ShapeDtypeStruct((n, m), jnp.float32),
  )(x, w, b)


def _mm1_stats(h, aggc, eps, w, b):
  n, din = h.shape
  _, m = w.shape
  kc = din // _LANES
  r = n // _BN

  def kern(h_ref, a_ref, e_ref, w_ref, b_ref, y_ref, s1_ref, s2_ref):
    i = pl.program_id(0)
    k = pl.program_id(1)
    z = (1.0 + e_ref[0, 0]) * h_ref[...] + a_ref[0]
    part = jnp.dot(z, w_ref[...], preferred_element_type=jnp.float32)

    @pl.when(k == 0)
    def _():
      y_ref[...] = part + b_ref[...]

    @pl.when(k > 0)
    def _():
      y_ref[...] += part

    @pl.when(k == kc - 1)
    def _():
      y = y_ref[...]
      cs = jnp.sum(y, axis=0, keepdims=True)
      cq = jnp.sum(y * y, axis=0, keepdims=True)

      @pl.when(i == 0)
      def _():
        s1_ref[...] = cs
        s2_ref[...] = cq

      @pl.when(i > 0)
      def _():
        s1_ref[...] += cs
        s2_ref[...] += cq

  return pl.pallas_call(
      kern,
      grid=(r, kc),
      in_specs=[
          pl.BlockSpec((_BN, _LANES), lambda i, k: (i, k)),
          pl.BlockSpec((1, _BN, _LANES), lambda i, k: (k, i, 0)),
          pl.BlockSpec((1, 1), lambda i, k: (0, 0)),
          pl.BlockSpec((_LANES, m), lambda i, k: (k, 0)),
          pl.BlockSpec((1, m), lambda i, k: (0, 0)),
      ],
      out_specs=[
          pl.BlockSpec((_BN, m), lambda i, k: (i, 0)),
          pl.BlockSpec((1, m), lambda i, k: (0, 0)),
          pl.BlockSpec((1, m), lambda i, k: (0, 0)),
      ],
      out_shape=[
          jax.ShapeDtypeStruct((n, m), jnp.float32),
          jax.ShapeDtypeStruct((1, m), jnp.float32),
          jax.ShapeDtypeStruct((1, m), jnp.float32),
      ],
  )(h, aggc, eps, w, b)


def _bn_mm_stats(y, s1, s2, g, bb, w, b):
  n, _ = y.shape
  k, m = w.shape
  r = n // _BN

  def kern(y_ref, s1_ref, s2_ref, g_ref, bb_ref, w_ref, b_ref,
           q_ref, q1_ref, q2_ref):
    i = pl.program_id(0)
    mean = s1_ref[...] / n
    var = s2_ref[...] / n - mean * mean
    inv = lax.rsqrt(var + 1e-5)
    t = jnp.maximum((y_ref[...] - mean) * (inv * g_ref[...]) + bb_ref[...],
                    0.0)
    q = jnp.dot(t, w_ref[...], preferred_element_type=jnp.float32) + b_ref[...]
    q_ref[...] = q
    cs = jnp.sum(q, axis=0, keepdims=True)
    cq = jnp.sum(q * q, axis=0, keepdims=True)

    @pl.when(i == 0)
    def _():
      q1_ref[...] = cs
      q2_ref[...] = cq

    @pl.when(i > 0)
    def _():
      q1_ref[...] += cs
      q2_ref[...] += cq

  return pl.pallas_call(
      kern,
      grid=(r,),
      in_specs=[
          pl.BlockSpec((_BN, k), lambda i: (i, 0)),
          pl.BlockSpec((1, k), lambda i: (0, 0)),
          pl.BlockSpec((1, k), lambda i: (0, 0)),
          pl.BlockSpec((1, k), lambda i: (0, 0)),
          pl.BlockSpec((1, k), lambda i: (0, 0)),
          pl.BlockSpec((k, m), lambda i: (0, 0)),
          pl.BlockSpec((1, m), lambda i: (0, 0)),
      ],
      out_specs=[
          pl.BlockSpec((_BN, m), lambda i: (i, 0)),
          pl.BlockSpec((1, m), lambda i: (0, 0)),
          pl.BlockSpec((1, m), lambda i: (0, 0)),
      ],
      out_shape=[
          jax.ShapeDtypeStruct((n, m), jnp.float32),
          jax.ShapeDtypeStruct((1, m), jnp.float32),
          jax.ShapeDtypeStruct((1, m), jnp.float32),
      ],
  )(y, s1, s2, g, bb, w, b)


def _bn_stats(q, s1, s2, g, bb):
  n, m = q.shape
  r = n // _BN

  def kern(q_ref, s1_ref, s2_ref, g_ref, bb_ref, t_ref, t1_ref, t2_ref):
    i = pl.program_id(0)
    mean = s1_ref[...] / n
    var = s2_ref[...] / n - mean * mean
    inv = lax.rsqrt(var + 1e-5)
    t = jnp.maximum((q_ref[...] - mean) * (inv * g_ref[...]) + bb_ref[...],
                    0.0)
    t_ref[...] = t
    cs = jnp.sum(t, axis=0, keepdims=True)
    cq = jnp.sum(t * t, axis=0, keepdims=True)

    @pl.when(i == 0)
    def _():
      t1_ref[...] = cs
      t2_ref[...] = cq

    @pl.when(i > 0)
    def _():
      t1_ref[...] += cs
      t2_ref[...] += cq

  return pl.pallas_call(
      kern,
      grid=(r,),
      in_specs=[
          pl.BlockSpec((_BN, m), lambda i: (i, 0)),
          pl.BlockSpec((1, m), lambda i: (0, 0)),
          pl.BlockSpec((1, m), lambda i: (0, 0)),
          pl.BlockSpec((1, m), lambda i: (0, 0)),
          pl.BlockSpec((1, m), lambda i: (0, 0)),
      ],
      out_specs=[
          pl.BlockSpec((_BN, m), lambda i: (i, 0)),
          pl.BlockSpec((1, m), lambda i: (0, 0)),
          pl.BlockSpec((1, m), lambda i: (0, 0)),
      ],
      out_shape=[
          jax.ShapeDtypeStruct((n, m), jnp.float32),
          jax.ShapeDtypeStruct((1, m), jnp.float32),
          jax.ShapeDtypeStruct((1, m), jnp.float32),
      ],
  )(q, s1, s2, g, bb)


def _bn_pred(t, s1, s2, g, bb, pw, score_in):
  n, m = t.shape
  _, mo = pw.shape
  r = n // _BN

  def kern(t_ref, s1_ref, s2_ref, g_ref, bb_ref, pw_ref, sc_ref,
           h_ref, so_ref):
    mean = s1_ref[...] / n
    var = s2_ref[...] / n - mean * mean
    inv = lax.rsqrt(var + 1e-5)
    h = jnp.maximum((t_ref[...] - mean) * (inv * g_ref[...]) + bb_ref[...],
                    0.0)
    h_ref[...] = h
    so_ref[...] = sc_ref[...] + jnp.dot(h, pw_ref[...],
                                        preferred_element_type=jnp.float32)

  return pl.pallas_call(
      kern,
      grid=(r,),
      in_specs=[
          pl.BlockSpec((_BN, m), lambda i: (i, 0)),
          pl.BlockSpec((1, m), lambda i: (0, 0)),
          pl.BlockSpec((1, m), lambda i: (0, 0)),
          pl.BlockSpec((1, m), lambda i: (0, 0)),
          pl.BlockSpec((1, m), lambda i: (0, 0)),
          pl.BlockSpec((m, mo), lambda i: (0, 0)),
          pl.BlockSpec((_BN, mo), lambda i: (i, 0)),
      ],
      out_specs=[
          pl.BlockSpec((_BN, m), lambda i: (i, 0)),
          pl.BlockSpec((_BN, mo), lambda i: (i, 0)),
      ],
      out_shape=[
          jax.ShapeDtypeStruct((n, m), jnp.float32),
          jax.ShapeDtypeStruct((n, mo), jnp.float32),
      ],
      input_output_aliases={6: 1},
  )(t, s1, s2, g, bb, pw, score_in)


def kernel(h, edge_index, params):
  n, din0 = h.shape
  e = edge_index.shape[1]

  nb = -(-e // (_NS * _BATCH))
  ep = _NS * nb * _BATCH
  npad = -(-n // (_NS * 8)) * (_NS * 8)
  pad_ids = jnp.arange(ep - e, dtype=jnp.int32)
  src = jnp.concatenate(
      [edge_index[0], pad_ids % n]).reshape(_NS, nb, _BATCH)
  dst = jnp.concatenate(
      [edge_index[1], n + pad_ids % (npad - n)]).reshape(_NS, nb, _BATCH)
  zeros = jnp.zeros((npad // _NS, _LANES), jnp.float32)

  def offset_src(nchunks):
    off = (jnp.arange(nchunks, dtype=jnp.int32) * n)[:, None, None, None]
    return (src[None] + off).reshape(nchunks * _NS, nb, _BATCH)

  src_by_nc = {}
  for i in range(3):
    nch = (din0 if i == 0 else 512) // _LANES
    if nch not in src_by_nc:
      src_by_nc[nch] = offset_src(nch)


  b_total = (params["pred0_b"] + params["pred1_b"] + params["pred2_b"]
             + params["pred3_b"]).reshape(1, -1)
  score = _mm_bias(h, params["pred0_W"], b_total)

  hcur = h
  for i in range(3):
    d = hcur.shape[1]
    nch = d // _LANES
    h2 = jnp.moveaxis(hcur.reshape(n, nch, _LANES), 1, 0).reshape(
        nch * n, _LANES)
    aggf = _segsum(h2, src_by_nc[nch], dst, zeros,
                   n=n, nchunks=nch, nb=nb)
    aggc = aggf.reshape(nch, npad, _LANES)

    eps = params[f"eps{i}"].reshape(1, 1)
    y, s1, s2 = _mm1_stats(hcur, aggc, eps,
                           params[f"mlp{i}_W1"],
                           params[f"mlp{i}_b1"].reshape(1, -1))
    q, q1, q2 = _bn_mm_stats(y, s1, s2,
                             params[f"mlp{i}_bng"].reshape(1, -1),
                             params[f"mlp{i}_bnb"].reshape(1, -1),
                             params[f"mlp{i}_W2"],
                             params[f"mlp{i}_b2"].reshape(1, -1))
    t, t1, t2 = _bn_stats(q, q1, q2,
                          params[f"apply{i}_bng"].reshape(1, -1),
                          params[f"apply{i}_bnb"].reshape(1, -1))
    hcur, score = _bn_pred(t, t1, t2,
                           params[f"out{i}_bng"].reshape(1, -1),
                           params[f"out{i}_bnb"].reshape(1, -1),
                           params[f"pred{i + 1}_W"], score)

  return score


# speedup vs baseline: 2.4373x; 1.3547x over previous
import functools

import jax
import jax.numpy as jnp
from jax import lax
from jax.experimental import pallas as pl
from jax.experimental.pallas import tpu as pltpu
from jax.experimental.pallas import tpu_sc as plsc

_NC = 2
_NS = 16
_LANES = 128
_BATCH = 128


@functools.partial(jax.jit, static_argnames=("n", "nchunks", "nb"))
def _segsum(h2, src_all, dstr, zeros, *, n, nchunks, nb):
  cpc = nchunks // _NC
  npad = -(-n // (_NS * 8)) * (_NS * 8)
  rpt = npad // _NS

  nbh = nb // 2

  mesh = plsc.VectorSubcoreMesh(core_axis_name="c", subcore_axis_name="s")

  @functools.partial(
      pl.kernel,
      out_type=jax.ShapeDtypeStruct((nchunks * npad, _LANES), jnp.float32),
      mesh=mesh,
      scratch_types=[
          pltpu.VMEM((nbh, _BATCH), jnp.int32),
          pltpu.VMEM((nbh, _BATCH), jnp.int32),
          pltpu.VMEM((_BATCH, _LANES), jnp.float32),
          pltpu.VMEM((_BATCH, _LANES), jnp.float32),
          pltpu.VMEM_SHARED((npad, _LANES), jnp.float32),
          pltpu.SemaphoreType.DMA,
          pltpu.SemaphoreType.DMA,
      ],
  )
  def seg_kernel(h_hbm, src_hbm, dst_hbm, zero_hbm, out_hbm,
                 src_v, dst_v, raw0, raw1, acc, sem0, sem1):
    c = lax.axis_index("c")
    s = lax.axis_index("s")
    raws = (raw0, raw1)
    sems = (sem0, sem1)

    def chunk_body(ci, carry):
      chunk = c + _NC * ci
      acc_off = pl.multiple_of(s * rpt, 8)
      pltpu.sync_copy(zero_hbm, acc.at[pl.ds(acc_off, rpt)])
      plsc.subcore_barrier()

      for half in range(2):
        pltpu.sync_copy(
            src_hbm.at[chunk * _NS + s, pl.ds(half * nbh, nbh)], src_v)
        pltpu.sync_copy(dst_hbm.at[s, pl.ds(half * nbh, nbh)], dst_v)
        desc = [None, None]
        desc[0] = pltpu.async_copy(h_hbm.at[src_v.at[0]], raw0, sem0)
        for b in range(nbh):
          cur = b % 2
          nxt = 1 - cur
          if b + 1 < nbh:
            desc[nxt] = pltpu.async_copy(h_hbm.at[src_v.at[b + 1]],
                                         raws[nxt], sems[nxt])
          desc[cur].wait()
          pltpu.sync_copy(raws[cur], acc.at[dst_v.at[b]], add=True)

      plsc.subcore_barrier()
      out_off = pl.multiple_of(chunk * npad + s * rpt, 8)
      pltpu.sync_copy(acc.at[pl.ds(acc_off, rpt)],
                      out_hbm.at[pl.ds(out_off, rpt)])
      return carry

    lax.fori_loop(0, cpc, chunk_body, 0)

  return seg_kernel(h2, src_all, dstr, zeros)


_BN = 2000


def _mm_bias(x, w, b):
  n, k = x.shape
  _, m = w.shape
  r = n // _BN

  def kern(x_ref, w_ref, b_ref, o_ref):
    o_ref[...] = jnp.dot(x_ref[...], w_ref[...],
                         preferred_element_type=jnp.float32) + b_ref[...]

  return pl.pallas_call(
      kern,
      grid=(r,),
      in_specs=[
          pl.BlockSpec((_BN, k), lambda i: (i, 0)),
          pl.BlockSpec((k, m), lambda i: (0, 0)),
          pl.BlockSpec((1, m), lambda i: (0, 0)),
      ],
      out_specs=pl.BlockSpec((_BN, m), lambda i: (i, 0)),
      out_shape=jax.ShapeDtypeStruct((n, m), jnp.float32),
  )(x, w, b)


def _mm1_stats(h, aggc, eps, w, b):
  n, din = h.shape
  _, m = w.shape
  kc = din // _LANES
  r = n // _BN

  def kern(h_ref, a_ref, e_ref, w_ref, b_ref, y_ref, s1_ref, s2_ref):
    i = pl.program_id(0)
    k = pl.program_id(1)
    z = (1.0 + e_ref[0, 0]) * h_ref[...] + a_ref[0]
    part = jnp.dot(z, w_ref[...], preferred_element_type=jnp.float32)

    @pl.when(k == 0)
    def _():
      y_ref[...] = part + b_ref[...]

    @pl.when(k > 0)
    def _():
      y_ref[...] += part

    @pl.when(k == kc - 1)
    def _():
      y = y_ref[...]
      cs = jnp.sum(y, axis=0, keepdims=True)
      cq = jnp.sum(y * y, axis=0, keepdims=True)

      @pl.when(i == 0)
      def _():
        s1_ref[...] = cs
        s2_ref[...] = cq

      @pl.when(i > 0)
      def _():
        s1_ref[...] += cs
        s2_ref[...] += cq

  return pl.pallas_call(
      kern,
      grid=(r, kc),
      in_specs=[
          pl.BlockSpec((_BN, _LANES), lambda i, k: (i, k)),
          pl.BlockSpec((1, _BN, _LANES), lambda i, k: (k, i, 0)),
          pl.BlockSpec((1, 1), lambda i, k: (0, 0)),
          pl.BlockSpec((_LANES, m), lambda i, k: (k, 0)),
          pl.BlockSpec((1, m), lambda i, k: (0, 0)),
      ],
      out_specs=[
          pl.BlockSpec((_BN, m), lambda i, k: (i, 0)),
          pl.BlockSpec((1, m), lambda i, k: (0, 0)),
          pl.BlockSpec((1, m), lambda i, k: (0, 0)),
      ],
      out_shape=[
          jax.ShapeDtypeStruct((n, m), jnp.float32),
          jax.ShapeDtypeStruct((1, m), jnp.float32),
          jax.ShapeDtypeStruct((1, m), jnp.float32),
      ],
  )(h, aggc, eps, w, b)


def _bn_mm_stats(y, s1, s2, g, bb, w, b):
  n, _ = y.shape
  k, m = w.shape
  r = n // _BN

  def kern(y_ref, s1_ref, s2_ref, g_ref, bb_ref, w_ref, b_ref,
           q_ref, q1_ref, q2_ref):
    i = pl.program_id(0)
    mean = s1_ref[...] / n
    var = s2_ref[...] / n - mean * mean
    inv = lax.rsqrt(var + 1e-5)
    t = jnp.maximum((y_ref[...] - mean) * (inv * g_ref[...]) + bb_ref[...],
                    0.0)
    q = jnp.dot(t, w_ref[...], preferred_element_type=jnp.float32) + b_ref[...]
    q_ref[...] = q
    cs = jnp.sum(q, axis=0, keepdims=True)
    cq = jnp.sum(q * q, axis=0, keepdims=True)

    @pl.when(i == 0)
    def _():
      q1_ref[...] = cs
      q2_ref[...] = cq

    @pl.when(i > 0)
    def _():
      q1_ref[...] += cs
      q2_ref[...] += cq

  return pl.pallas_call(
      kern,
      grid=(r,),
      in_specs=[
          pl.BlockSpec((_BN, k), lambda i: (i, 0)),
          pl.BlockSpec((1, k), lambda i: (0, 0)),
          pl.BlockSpec((1, k), lambda i: (0, 0)),
          pl.BlockSpec((1, k), lambda i: (0, 0)),
          pl.BlockSpec((1, k), lambda i: (0, 0)),
          pl.BlockSpec((k, m), lambda i: (0, 0)),
          pl.BlockSpec((1, m), lambda i: (0, 0)),
      ],
      out_specs=[
          pl.BlockSpec((_BN, m), lambda i: (i, 0)),
          pl.BlockSpec((1, m), lambda i: (0, 0)),
          pl.BlockSpec((1, m), lambda i: (0, 0)),
      ],
      out_shape=[
          jax.ShapeDtypeStruct((n, m), jnp.float32),
          jax.ShapeDtypeStruct((1, m), jnp.float32),
          jax.ShapeDtypeStruct((1, m), jnp.float32),
      ],
  )(y, s1, s2, g, bb, w, b)


def _bn_stats(q, s1, s2, g, bb):
  n, m = q.shape
  r = n // _BN

  def kern(q_ref, s1_ref, s2_ref, g_ref, bb_ref, t_ref, t1_ref, t2_ref):
    i = pl.program_id(0)
    mean = s1_ref[...] / n
    var = s2_ref[...] / n - mean * mean
    inv = lax.rsqrt(var + 1e-5)
    t = jnp.maximum((q_ref[...] - mean) * (inv * g_ref[...]) + bb_ref[...],
                    0.0)
    t_ref[...] = t
    cs = jnp.sum(t, axis=0, keepdims=True)
    cq = jnp.sum(t * t, axis=0, keepdims=True)

    @pl.when(i == 0)
    def _():
      t1_ref[...] = cs
      t2_ref[...] = cq

    @pl.when(i > 0)
    def _():
      t1_ref[...] += cs
      t2_ref[...] += cq

  return pl.pallas_call(
      kern,
      grid=(r,),
      in_specs=[
          pl.BlockSpec((_BN, m), lambda i: (i, 0)),
          pl.BlockSpec((1, m), lambda i: (0, 0)),
          pl.BlockSpec((1, m), lambda i: (0, 0)),
          pl.BlockSpec((1, m), lambda i: (0, 0)),
          pl.BlockSpec((1, m), lambda i: (0, 0)),
      ],
      out_specs=[
          pl.BlockSpec((_BN, m), lambda i: (i, 0)),
          pl.BlockSpec((1, m), lambda i: (0, 0)),
          pl.BlockSpec((1, m), lambda i: (0, 0)),
      ],
      out_shape=[
          jax.ShapeDtypeStruct((n, m), jnp.float32),
          jax.ShapeDtypeStruct((1, m), jnp.float32),
          jax.ShapeDtypeStruct((1, m), jnp.float32),
      ],
  )(q, s1, s2, g, bb)


def _bn_pred(t, s1, s2, g, bb, pw, score_in):
  n, m = t.shape
  _, mo = pw.shape
  r = n // _BN

  def kern(t_ref, s1_ref, s2_ref, g_ref, bb_ref, pw_ref, sc_ref,
           h_ref, so_ref):
    mean = s1_ref[...] / n
    var = s2_ref[...] / n - mean * mean
    inv = lax.rsqrt(var + 1e-5)
    h = jnp.maximum((t_ref[...] - mean) * (inv * g_ref[...]) + bb_ref[...],
                    0.0)
    h_ref[...] = h
    so_ref[...] = sc_ref[...] + jnp.dot(h, pw_ref[...],
                                        preferred_element_type=jnp.float32)

  return pl.pallas_call(
      kern,
      grid=(r,),
      in_specs=[
          pl.BlockSpec((_BN, m), lambda i: (i, 0)),
          pl.BlockSpec((1, m), lambda i: (0, 0)),
          pl.BlockSpec((1, m), lambda i: (0, 0)),
          pl.BlockSpec((1, m), lambda i: (0, 0)),
          pl.BlockSpec((1, m), lambda i: (0, 0)),
          pl.BlockSpec((m, mo), lambda i: (0, 0)),
          pl.BlockSpec((_BN, mo), lambda i: (i, 0)),
      ],
      out_specs=[
          pl.BlockSpec((_BN, m), lambda i: (i, 0)),
          pl.BlockSpec((_BN, mo), lambda i: (i, 0)),
      ],
      out_shape=[
          jax.ShapeDtypeStruct((n, m), jnp.float32),
          jax.ShapeDtypeStruct((n, mo), jnp.float32),
      ],
      input_output_aliases={6: 1},
  )(t, s1, s2, g, bb, pw, score_in)


def kernel(h, edge_index, params):
  n, din0 = h.shape
  e = edge_index.shape[1]

  nb = -(-e // (_NS * _BATCH * 16)) * 16
  ep = _NS * nb * _BATCH
  npad = -(-n // (_NS * 8)) * (_NS * 8)
  pad_ids = jnp.arange(ep - e, dtype=jnp.int32)
  src = jnp.concatenate(
      [edge_index[0], pad_ids % n]).reshape(_NS, nb, _BATCH)
  dst = jnp.concatenate(
      [edge_index[1], n + pad_ids % (npad - n)]).reshape(_NS, nb, _BATCH)
  zeros = jnp.zeros((npad // _NS, _LANES), jnp.float32)

  def offset_src(nchunks):
    off = (jnp.arange(nchunks, dtype=jnp.int32) * n)[:, None, None, None]
    return (src[None] + off).reshape(nchunks * _NS, nb, _BATCH)

  src_by_nc = {}
  for i in range(3):
    nch = (din0 if i == 0 else 512) // _LANES
    if nch not in src_by_nc:
      src_by_nc[nch] = offset_src(nch)


  b_total = (params["pred0_b"] + params["pred1_b"] + params["pred2_b"]
             + params["pred3_b"]).reshape(1, -1)
  score = _mm_bias(h, params["pred0_W"], b_total)

  hcur = h
  for i in range(3):
    d = hcur.shape[1]
    nch = d // _LANES
    h2 = jnp.moveaxis(hcur.reshape(n, nch, _LANES), 1, 0).reshape(
        nch * n, _LANES)
    aggf = _segsum(h2, src_by_nc[nch], dst, zeros,
                   n=n, nchunks=nch, nb=nb)
    aggc = aggf.reshape(nch, npad, _LANES)

    eps = params[f"eps{i}"].reshape(1, 1)
    y, s1, s2 = _mm1_stats(hcur, aggc, eps,
                           params[f"mlp{i}_W1"],
                           params[f"mlp{i}_b1"].reshape(1, -1))
    q, q1, q2 = _bn_mm_stats(y, s1, s2,
                             params[f"mlp{i}_bng"].reshape(1, -1),
                             params[f"mlp{i}_bnb"].reshape(1, -1),
                             params[f"mlp{i}_W2"],
                             params[f"mlp{i}_b2"].reshape(1, -1))
    t, t1, t2 = _bn_stats(q, q1, q2,
                          params[f"apply{i}_bng"].reshape(1, -1),
                          params[f"apply{i}_bnb"].reshape(1, -1))
    hcur, score = _bn_pred(t, t1, t2,
                           params[f"out{i}_bng"].reshape(1, -1),
                           params[f"out{i}_bnb"].reshape(1, -1),
                           params[f"pred{i + 1}_W"], score)

  return score
